# baseline (speedup 1.0000x reference)
import functools

import jax
import jax.numpy as jnp
from jax import lax
from jax.experimental import pallas as pl
from jax.experimental.pallas import tpu as pltpu
from jax.experimental.pallas import tpu_sc as plsc

N_NODES = 10000
N_EDGES = 320000
FEAT = 128

NC = 2
NS = 16
NW = NC * NS
EDGES_PER_W = N_EDGES // NW
CHUNK = 125
GCH = 20
NG = EDGES_PER_W // (CHUNK * GCH)
NBUF = 2
ROW_BLK = 200
ZROWS = 40
N_ROW_BLK = N_NODES // ROW_BLK


def _scatter_body(t_hbm, edge_hbm, out_hbm,
                  acc, src_v0, src_v1, dst_v0, dst_v1, buf0, buf1, zbuf,
                  gsem0, gsem1, ssem0, ssem1, isem0, isem1):
    c = lax.axis_index("c")
    s = lax.axis_index("s")
    wid = c * NS + s

    bufs = (buf0, buf1)
    gsems = (gsem0, gsem1)
    ssems = (ssem0, ssem1)
    src_vs = (src_v0, src_v1)
    dst_vs = (dst_v0, dst_v1)
    isems = (isem0, isem1)

    def idx_refs(j):
        bank = (j // GCH) % 2
        return src_vs[bank].at[j % GCH], dst_vs[bank].at[j % GCH]

    def idx_start(g):
        bank = g % 2
        pltpu.async_copy(edge_hbm.at[0, wid, g], src_vs[bank], isems[bank])
        pltpu.async_copy(edge_hbm.at[1, wid, g], dst_vs[bank], isems[bank])

    def idx_wait(g):
        bank = g % 2
        pltpu.make_async_copy(edge_hbm.at[0, wid, g], src_vs[bank],
                              isems[bank]).wait()
        pltpu.make_async_copy(edge_hbm.at[1, wid, g], dst_vs[bank],
                              isems[bank]).wait()

    def g_start(j):
        b = j % NBUF
        sv, _ = idx_refs(j)
        pltpu.async_copy(t_hbm.at[sv], bufs[b], gsems[b])

    def g_wait(j):
        b = j % NBUF
        sv, _ = idx_refs(j)
        pltpu.make_async_copy(t_hbm.at[sv], bufs[b], gsems[b]).wait()

    def s_start(j):
        b = j % NBUF
        _, dv = idx_refs(j)
        pltpu.async_copy(bufs[b], acc.at[dv], ssems[b], add=True)

    def s_wait(j):
        b = j % NBUF
        _, dv = idx_refs(j)
        pltpu.make_async_copy(bufs[b], acc.at[dv], ssems[b]).wait()

    idx_start(0)
    idx_wait(0)
    g_start(0)
    g_start(1)
    idx_start(1)

    zero = jnp.zeros((16,), jnp.float32)

    @pl.loop(0, ZROWS)
    def _zero_rows(i):
        for v in range(FEAT // 16):
            zbuf[i, pl.ds(v * 16, 16)] = zero

    for k in range((N_ROW_BLK + NS - 1) // NS):
        m = s + NS * k

        @pl.when(m < N_ROW_BLK)
        def _():
            for i in range(ROW_BLK // ZROWS):
                r = pl.multiple_of(m * ROW_BLK + i * ZROWS, 8)
                pltpu.sync_copy(zbuf, acc.at[pl.ds(r, ZROWS)])

    plsc.subcore_barrier()

    NCH = NG * GCH
    for j in range(NCH):
        g_wait(j)
        s_start(j)
        s_wait(j)
        if j + 2 < NCH:
            nj = j + 2
            if nj % GCH == 0:
                idx_wait(nj // GCH)
            g_start(nj)
            if nj % GCH == 1 and nj // GCH + 1 < NG:
                idx_start(nj // GCH + 1)

    plsc.subcore_barrier()

    for k in range((N_ROW_BLK + NS - 1) // NS):
        m = s + NS * k

        @pl.when(m < N_ROW_BLK)
        def _():
            r = pl.multiple_of(m * ROW_BLK, 8)
            pltpu.sync_copy(acc.at[pl.ds(r, ROW_BLK)],
                            out_hbm.at[c, pl.ds(r, ROW_BLK)])


_IDX_SHAPE = (2, NW, NG, GCH, CHUNK)


@jax.jit
def _sc_scatter(t, edges):
    mesh = plsc.VectorSubcoreMesh(core_axis_name="c", subcore_axis_name="s")
    return pl.kernel(
        _scatter_body,
        out_type=jax.ShapeDtypeStruct((NC, N_NODES, FEAT), jnp.float32),
        mesh=mesh,
        scratch_types=[
            pltpu.VMEM_SHARED((N_NODES, FEAT), jnp.float32),
            pltpu.VMEM((GCH, CHUNK), jnp.int32),
            pltpu.VMEM((GCH, CHUNK), jnp.int32),
            pltpu.VMEM((GCH, CHUNK), jnp.int32),
            pltpu.VMEM((GCH, CHUNK), jnp.int32),
            pltpu.VMEM((CHUNK, FEAT), jnp.float32),
            pltpu.VMEM((CHUNK, FEAT), jnp.float32),
            pltpu.VMEM((ZROWS, FEAT), jnp.float32),
            pltpu.SemaphoreType.DMA,
            pltpu.SemaphoreType.DMA,
            pltpu.SemaphoreType.DMA,
            pltpu.SemaphoreType.DMA,
            pltpu.SemaphoreType.DMA,
            pltpu.SemaphoreType.DMA,
        ],
    )(t, edges)


def _mm1_body(x_ref, w_ref, b_ref, o_ref):
    o_ref[...] = (
        jnp.dot(x_ref[...].astype(jnp.bfloat16),
                w_ref[...].astype(jnp.bfloat16),
                preferred_element_type=jnp.float32)
        + b_ref[...]
    )


def _mm2_body(p_ref, w_ref, b_ref, o_ref):
    ps = (p_ref[0] + p_ref[1]).astype(jnp.bfloat16)
    w = (w_ref[:FEAT, :] + w_ref[FEAT:, :]).astype(jnp.bfloat16)
    o_ref[...] = jnp.dot(ps, w, preferred_element_type=jnp.float32) + b_ref[...]


def _combine_body(q_ref, o_ref):
    sres = q_ref[0] + q_ref[1]
    o_ref[:, :FEAT] = sres
    o_ref[:, FEAT:] = sres


_ROWS_BLK = 2000


@jax.jit
def _tc_mm1(x, W1, b1):
    return pl.pallas_call(
        _mm1_body,
        grid=(N_NODES // _ROWS_BLK,),
        in_specs=[
            pl.BlockSpec((_ROWS_BLK, FEAT), lambda i: (i, 0)),
            pl.BlockSpec((FEAT, FEAT), lambda i: (0, 0)),
            pl.BlockSpec((FEAT,), lambda i: (0,)),
        ],
        out_specs=pl.BlockSpec((_ROWS_BLK, FEAT), lambda i: (i, 0)),
        out_shape=jax.ShapeDtypeStruct((N_NODES, FEAT), jnp.float32),
    )(x, W1, b1)


@jax.jit
def _tc_mm2(p, W2, b2):
    return pl.pallas_call(
        _mm2_body,
        grid=(N_NODES // _ROWS_BLK,),
        in_specs=[
            pl.BlockSpec((NC, _ROWS_BLK, FEAT), lambda i: (0, i, 0)),
            pl.BlockSpec((2 * FEAT, FEAT), lambda i: (0, 0)),
            pl.BlockSpec((FEAT,), lambda i: (0,)),
        ],
        out_specs=pl.BlockSpec((_ROWS_BLK, FEAT), lambda i: (i, 0)),
        out_shape=jax.ShapeDtypeStruct((N_NODES, FEAT), jnp.float32),
    )(p, W2, b2)


@jax.jit
def _tc_combine(q):
    return pl.pallas_call(
        _combine_body,
        grid=(N_NODES // _ROWS_BLK,),
        in_specs=[pl.BlockSpec((NC, _ROWS_BLK, FEAT), lambda i: (0, i, 0))],
        out_specs=pl.BlockSpec((_ROWS_BLK, 2 * FEAT), lambda i: (i, 0)),
        out_shape=jax.ShapeDtypeStruct((N_NODES, 2 * FEAT), jnp.float32),
    )(q)


def kernel(x, edge_index, W1, b1, W2, b2):
    edges = edge_index.astype(jnp.int32).reshape(_IDX_SHAPE)

    t1 = _tc_mm1(x, W1, b1)
    p = _sc_scatter(t1, edges)
    t2 = _tc_mm2(p, W2, b2)
    q = _sc_scatter(t2, edges)
    return _tc_combine(q)


# speedup vs baseline: 1.0018x; 1.0018x over previous
import jax
import jax.numpy as jnp
from jax import lax
from jax.experimental import pallas as pl
from jax.experimental.pallas import tpu as pltpu
from jax.experimental.pallas import tpu_sc as plsc

N_NODES = 10000
N_EDGES = 320000
FEAT = 128

NC = 2
NS = 16
NW = NC * NS
EDGES_PER_W = N_EDGES // NW
CHUNK = 125
GCH = 20
NG = EDGES_PER_W // (CHUNK * GCH)
NBUF = 2
ROW_BLK = 200
ZROWS = 40
N_ROW_BLK = N_NODES // ROW_BLK


def _scatter_body(t_hbm, edge_hbm, out_hbm,
                  acc, src_v0, src_v1, dst_v0, dst_v1, buf0, buf1, zbuf,
                  gsem0, gsem1, ssem0, ssem1, isem0, isem1):
    c = lax.axis_index("c")
    s = lax.axis_index("s")
    wid = c * NS + s

    bufs = (buf0, buf1)
    gsems = (gsem0, gsem1)
    ssems = (ssem0, ssem1)
    src_vs = (src_v0, src_v1)
    dst_vs = (dst_v0, dst_v1)
    isems = (isem0, isem1)

    def idx_refs(j):
        bank = (j // GCH) % 2
        return src_vs[bank].at[j % GCH], dst_vs[bank].at[j % GCH]

    def idx_start(g):
        bank = g % 2
        pltpu.async_copy(edge_hbm.at[0, wid, g], src_vs[bank], isems[bank])
        pltpu.async_copy(edge_hbm.at[1, wid, g], dst_vs[bank], isems[bank])

    def idx_wait(g):
        bank = g % 2
        pltpu.make_async_copy(edge_hbm.at[0, wid, g], src_vs[bank],
                              isems[bank]).wait()
        pltpu.make_async_copy(edge_hbm.at[1, wid, g], dst_vs[bank],
                              isems[bank]).wait()

    def g_start(j):
        b = j % NBUF
        sv, _ = idx_refs(j)
        pltpu.async_copy(t_hbm.at[sv], bufs[b], gsems[b])

    def g_wait(j):
        b = j % NBUF
        sv, _ = idx_refs(j)
        pltpu.make_async_copy(t_hbm.at[sv], bufs[b], gsems[b]).wait()

    def s_start(j):
        b = j % NBUF
        _, dv = idx_refs(j)
        pltpu.async_copy(bufs[b], acc.at[dv], ssems[b], add=True)

    def s_wait(j):
        b = j % NBUF
        _, dv = idx_refs(j)
        pltpu.make_async_copy(bufs[b], acc.at[dv], ssems[b]).wait()

    idx_start(0)
    idx_wait(0)
    g_start(0)
    g_start(1)
    idx_start(1)

    zero = jnp.zeros((16,), jnp.float32)

    @pl.loop(0, ZROWS)
    def _zero_rows(i):
        for v in range(FEAT // 16):
            zbuf[i, pl.ds(v * 16, 16)] = zero

    for k in range((N_ROW_BLK + NS - 1) // NS):
        m = s + NS * k

        @pl.when(m < N_ROW_BLK)
        def _():
            for i in range(ROW_BLK // ZROWS):
                r = pl.multiple_of(m * ROW_BLK + i * ZROWS, 8)
                pltpu.sync_copy(zbuf, acc.at[pl.ds(r, ZROWS)])

    plsc.subcore_barrier()

    NCH = NG * GCH
    for j in range(NCH):
        g_wait(j)
        s_start(j)
        s_wait(j)
        if j + 2 < NCH:
            nj = j + 2
            if nj % GCH == 0:
                idx_wait(nj // GCH)
            g_start(nj)
            if nj % GCH == 1 and nj // GCH + 1 < NG:
                idx_start(nj // GCH + 1)

    plsc.subcore_barrier()

    for k in range((N_ROW_BLK + NS - 1) // NS):
        m = s + NS * k

        @pl.when(m < N_ROW_BLK)
        def _():
            r = pl.multiple_of(m * ROW_BLK, 8)
            pltpu.sync_copy(acc.at[pl.ds(r, ROW_BLK)],
                            out_hbm.at[c, pl.ds(r, ROW_BLK)])


_IDX_SHAPE = (2, NW, NG, GCH, CHUNK)


@jax.jit
def _sc_scatter(t, edges):
    mesh = plsc.VectorSubcoreMesh(core_axis_name="c", subcore_axis_name="s")
    return pl.kernel(
        _scatter_body,
        out_type=jax.ShapeDtypeStruct((NC, N_NODES, FEAT), jnp.float32),
        mesh=mesh,
        scratch_types=[
            pltpu.VMEM_SHARED((N_NODES, FEAT), jnp.float32),
            pltpu.VMEM((GCH, CHUNK), jnp.int32),
            pltpu.VMEM((GCH, CHUNK), jnp.int32),
            pltpu.VMEM((GCH, CHUNK), jnp.int32),
            pltpu.VMEM((GCH, CHUNK), jnp.int32),
            pltpu.VMEM((CHUNK, FEAT), jnp.float32),
            pltpu.VMEM((CHUNK, FEAT), jnp.float32),
            pltpu.VMEM((ZROWS, FEAT), jnp.float32),
            pltpu.SemaphoreType.DMA,
            pltpu.SemaphoreType.DMA,
            pltpu.SemaphoreType.DMA,
            pltpu.SemaphoreType.DMA,
            pltpu.SemaphoreType.DMA,
            pltpu.SemaphoreType.DMA,
        ],
    )(t, edges)


def _mm1_body(x_ref, w_ref, b_ref, o_ref):
    o_ref[...] = (
        jnp.dot(x_ref[...], w_ref[...], preferred_element_type=jnp.float32)
        + b_ref[...]
    )


def _mm2_body(p_ref, w_ref, b_ref, o_ref):
    ps = p_ref[0] + p_ref[1]
    w = w_ref[:FEAT, :] + w_ref[FEAT:, :]
    o_ref[...] = jnp.dot(ps, w, preferred_element_type=jnp.float32) + b_ref[...]


def _combine_body(q_ref, o_ref):
    sres = q_ref[0] + q_ref[1]
    o_ref[:, :FEAT] = sres
    o_ref[:, FEAT:] = sres


_ROWS_BLK = 2000


@jax.jit
def _tc_mm1(x, W1, b1):
    return pl.pallas_call(
        _mm1_body,
        grid=(N_NODES // _ROWS_BLK,),
        in_specs=[
            pl.BlockSpec((_ROWS_BLK, FEAT), lambda i: (i, 0)),
            pl.BlockSpec((FEAT, FEAT), lambda i: (0, 0)),
            pl.BlockSpec((FEAT,), lambda i: (0,)),
        ],
        out_specs=pl.BlockSpec((_ROWS_BLK, FEAT), lambda i: (i, 0)),
        out_shape=jax.ShapeDtypeStruct((N_NODES, FEAT), jnp.float32),
    )(x, W1, b1)


@jax.jit
def _tc_mm2(p, W2, b2):
    return pl.pallas_call(
        _mm2_body,
        grid=(N_NODES // _ROWS_BLK,),
        in_specs=[
            pl.BlockSpec((NC, _ROWS_BLK, FEAT), lambda i: (0, i, 0)),
            pl.BlockSpec((2 * FEAT, FEAT), lambda i: (0, 0)),
            pl.BlockSpec((FEAT,), lambda i: (0,)),
        ],
        out_specs=pl.BlockSpec((_ROWS_BLK, FEAT), lambda i: (i, 0)),
        out_shape=jax.ShapeDtypeStruct((N_NODES, FEAT), jnp.float32),
    )(p, W2, b2)


@jax.jit
def _tc_combine(q):
    return pl.pallas_call(
        _combine_body,
        grid=(N_NODES // _ROWS_BLK,),
        in_specs=[pl.BlockSpec((NC, _ROWS_BLK, FEAT), lambda i: (0, i, 0))],
        out_specs=pl.BlockSpec((_ROWS_BLK, 2 * FEAT), lambda i: (i, 0)),
        out_shape=jax.ShapeDtypeStruct((N_NODES, 2 * FEAT), jnp.float32),
    )(q)


def kernel(x, edge_index, W1, b1, W2, b2):
    edges = edge_index.astype(jnp.int32).reshape(_IDX_SHAPE)

    t1 = _tc_mm1(x, W1, b1)
    p = _sc_scatter(t1, edges)
    t2 = _tc_mm2(p, W2, b2)
    q = _sc_scatter(t2, edges)
    return _tc_combine(q)


# speedup vs baseline: 1.0272x; 1.0253x over previous
import jax
import jax.numpy as jnp
from jax import lax
from jax.experimental import pallas as pl
from jax.experimental.pallas import tpu as pltpu
from jax.experimental.pallas import tpu_sc as plsc

N_NODES = 10000
N_EDGES = 320000
FEAT = 128

NC = 2
NS = 16
NW = NC * NS
EDGES_PER_W = N_EDGES // NW
CHUNK = 125
GCH = 20
NG = EDGES_PER_W // (CHUNK * GCH)
NBUF = 2
ROW_BLK = 200
ZROWS = 40
N_ROW_BLK = N_NODES // ROW_BLK


def _scatter_body(t_hbm, edge_hbm, out_hbm,
                  acc, src_v0, src_v1, dst_v0, dst_v1, buf0, buf1, zbuf,
                  gsem0, gsem1, ssem0, ssem1, isem0, isem1):
    c = lax.axis_index("c")
    s = lax.axis_index("s")
    wid = c * NS + s

    bufs = (buf0, buf1)
    gsems = (gsem0, gsem1)
    ssems = (ssem0, ssem1)
    src_vs = (src_v0, src_v1)
    dst_vs = (dst_v0, dst_v1)
    isems = (isem0, isem1)

    def idx_refs(j):
        bank = (j // GCH) % 2
        return src_vs[bank].at[j % GCH], dst_vs[bank].at[j % GCH]

    def idx_start(g):
        bank = g % 2
        pltpu.async_copy(edge_hbm.at[0, wid, g], src_vs[bank], isems[bank])
        pltpu.async_copy(edge_hbm.at[1, wid, g], dst_vs[bank], isems[bank])

    def idx_wait(g):
        bank = g % 2
        pltpu.make_async_copy(edge_hbm.at[0, wid, g], src_vs[bank],
                              isems[bank]).wait()
        pltpu.make_async_copy(edge_hbm.at[1, wid, g], dst_vs[bank],
                              isems[bank]).wait()

    def g_start(j):
        b = j % NBUF
        sv, _ = idx_refs(j)
        pltpu.async_copy(t_hbm.at[sv], bufs[b], gsems[b])

    def g_wait(j):
        b = j % NBUF
        sv, _ = idx_refs(j)
        pltpu.make_async_copy(t_hbm.at[sv], bufs[b], gsems[b]).wait()

    def s_start(j):
        b = j % NBUF
        _, dv = idx_refs(j)
        pltpu.async_copy(bufs[b], acc.at[dv], ssems[b], add=True)

    def s_wait(j):
        b = j % NBUF
        _, dv = idx_refs(j)
        pltpu.make_async_copy(bufs[b], acc.at[dv], ssems[b]).wait()

    idx_start(0)
    idx_wait(0)
    g_start(0)
    g_start(1)
    idx_start(1)

    zero = jnp.zeros((16,), jnp.float32)

    @pl.loop(0, ZROWS)
    def _zero_rows(i):
        for v in range(FEAT // 16):
            zbuf[i, pl.ds(v * 16, 16)] = zero

    for k in range((N_ROW_BLK + NS - 1) // NS):
        m = s + NS * k

        @pl.when(m < N_ROW_BLK)
        def _():
            for i in range(ROW_BLK // ZROWS):
                r = pl.multiple_of(m * ROW_BLK + i * ZROWS, 8)
                pltpu.sync_copy(zbuf, acc.at[pl.ds(r, ZROWS)])

    plsc.subcore_barrier()

    NCH = NG * GCH
    for j in range(NCH):
        g_wait(j)
        s_start(j)
        s_wait(j)
        if j + 2 < NCH:
            nj = j + 2
            if nj % GCH == 0:
                idx_wait(nj // GCH)
            g_start(nj)
            if nj % GCH == 1 and nj // GCH + 1 < NG:
                idx_start(nj // GCH + 1)

    plsc.subcore_barrier()

    for k in range((N_ROW_BLK + NS - 1) // NS):
        m = s + NS * k

        @pl.when(m < N_ROW_BLK)
        def _():
            r = pl.multiple_of(m * ROW_BLK, 8)
            pltpu.sync_copy(acc.at[pl.ds(r, ROW_BLK)],
                            out_hbm.at[c, pl.ds(r, ROW_BLK)])


_IDX_SHAPE = (2, NW, NG, GCH, CHUNK)


@jax.jit
def _sc_scatter(t, edges):
    mesh = plsc.VectorSubcoreMesh(core_axis_name="c", subcore_axis_name="s")
    return pl.kernel(
        _scatter_body,
        out_type=jax.ShapeDtypeStruct((NC, N_NODES, FEAT), jnp.float32),
        mesh=mesh,
        scratch_types=[
            pltpu.VMEM_SHARED((N_NODES, FEAT), jnp.float32),
            pltpu.VMEM((GCH, CHUNK), jnp.int32),
            pltpu.VMEM((GCH, CHUNK), jnp.int32),
            pltpu.VMEM((GCH, CHUNK), jnp.int32),
            pltpu.VMEM((GCH, CHUNK), jnp.int32),
            pltpu.VMEM((CHUNK, FEAT), jnp.float32),
            pltpu.VMEM((CHUNK, FEAT), jnp.float32),
            pltpu.VMEM((ZROWS, FEAT), jnp.float32),
            pltpu.SemaphoreType.DMA,
            pltpu.SemaphoreType.DMA,
            pltpu.SemaphoreType.DMA,
            pltpu.SemaphoreType.DMA,
            pltpu.SemaphoreType.DMA,
            pltpu.SemaphoreType.DMA,
        ],
    )(t, edges)


def _mm1_body(x_ref, w_ref, b_ref, o_ref):
    o_ref[...] = (
        jnp.dot(x_ref[...], w_ref[...], preferred_element_type=jnp.float32)
        + b_ref[...]
    )


def _mm2_body(p_ref, w_ref, b_ref, o_ref):
    ps = p_ref[0] + p_ref[1]
    w = w_ref[:FEAT, :] + w_ref[FEAT:, :]
    o_ref[...] = jnp.dot(ps, w, preferred_element_type=jnp.float32) + b_ref[...]


def _combine_body(q_ref, o_ref):
    sres = q_ref[0] + q_ref[1]
    o_ref[:, :FEAT] = sres
    o_ref[:, FEAT:] = sres


_ROWS_BLK = 5000


@jax.jit
def _tc_mm1(x, W1, b1):
    return pl.pallas_call(
        _mm1_body,
        grid=(N_NODES // _ROWS_BLK,),
        in_specs=[
            pl.BlockSpec((_ROWS_BLK, FEAT), lambda i: (i, 0)),
            pl.BlockSpec((FEAT, FEAT), lambda i: (0, 0)),
            pl.BlockSpec((FEAT,), lambda i: (0,)),
        ],
        out_specs=pl.BlockSpec((_ROWS_BLK, FEAT), lambda i: (i, 0)),
        out_shape=jax.ShapeDtypeStruct((N_NODES, FEAT), jnp.float32),
    )(x, W1, b1)


@jax.jit
def _tc_mm2(p, W2, b2):
    return pl.pallas_call(
        _mm2_body,
        grid=(N_NODES // _ROWS_BLK,),
        in_specs=[
            pl.BlockSpec((NC, _ROWS_BLK, FEAT), lambda i: (0, i, 0)),
            pl.BlockSpec((2 * FEAT, FEAT), lambda i: (0, 0)),
            pl.BlockSpec((FEAT,), lambda i: (0,)),
        ],
        out_specs=pl.BlockSpec((_ROWS_BLK, FEAT), lambda i: (i, 0)),
        out_shape=jax.ShapeDtypeStruct((N_NODES, FEAT), jnp.float32),
    )(p, W2, b2)


@jax.jit
def _tc_combine(q):
    return pl.pallas_call(
        _combine_body,
        grid=(N_NODES // _ROWS_BLK,),
        in_specs=[pl.BlockSpec((NC, _ROWS_BLK, FEAT), lambda i: (0, i, 0))],
        out_specs=pl.BlockSpec((_ROWS_BLK, 2 * FEAT), lambda i: (i, 0)),
        out_shape=jax.ShapeDtypeStruct((N_NODES, 2 * FEAT), jnp.float32),
    )(q)


def kernel(x, edge_index, W1, b1, W2, b2):
    edges = edge_index.astype(jnp.int32).reshape(_IDX_SHAPE)

    t1 = _tc_mm1(x, W1, b1)
    p = _sc_scatter(t1, edges)
    t2 = _tc_mm2(p, W2, b2)
    q = _sc_scatter(t2, edges)
    return _tc_combine(q)
